# baseline (speedup 1.0000x reference)
import functools

import jax
import jax.numpy as jnp
from jax import lax
from jax.experimental import pallas as pl
from jax.experimental.pallas import tpu as pltpu
from jax.experimental.pallas import tpu_sc as plsc

LANES = 16
NC = 2
NS = 16
NW = NC * NS
BCHUNK = 8
OBUF = 2


@functools.lru_cache(maxsize=None)
def _build_sc_kernel(B: int, F: int, D: int):
    per_wb = B // NW
    per_w = per_wb * F
    chunk = BCHUNK * F
    groups = chunk // LANES
    assert chunk % LANES == 0
    n_chunks = per_wb // BCHUNK
    assert n_chunks % OBUF == 0
    mesh = plsc.VectorSubcoreMesh(core_axis_name="c", subcore_axis_name="s")

    @functools.partial(
        pl.kernel,
        mesh=mesh,
        out_type=jax.ShapeDtypeStruct((B, F, D), jnp.float32),
        scratch_types=(
            [
                pltpu.VMEM((per_w,), jnp.int32),
                pltpu.VMEM((per_w,), jnp.float32),
                pltpu.VMEM((groups, LANES * D), jnp.float32),
                pltpu.VMEM((groups, LANES * D), jnp.float32),
                pltpu.VMEM((OBUF, BCHUNK, F, D), jnp.float32),
            ]
            + [pltpu.SemaphoreType.DMA((groups,)),
               pltpu.SemaphoreType.DMA((groups,))]
            + [pltpu.SemaphoreType.DMA] * OBUF
        ),
    )
    def sc_kernel(tok_hbm, inp_hbm, emb_hbm, bias_hbm, out_hbm,
                  idx_v, inp_v, emb_v, bias_v, out_v, e_sem, b_sem, *o_sem):
        wid = lax.axis_index("s") * NC + lax.axis_index("c")
        base = wid * per_w
        base_b = wid * per_wb

        pltpu.sync_copy(tok_hbm.at[pl.ds(base, per_w)], idx_v)
        pltpu.sync_copy(inp_hbm.at[pl.ds(base, per_w)], inp_v)

        def fire_group(c, gg):
            tokv = idx_v[pl.ds(c * chunk + gg * LANES, LANES)]
            for r in range(LANES):
                t = tokv[r]
                pltpu.async_copy(emb_hbm.at[t],
                                 emb_v.at[gg, pl.ds(r * D, D)], e_sem.at[gg])
                pltpu.async_copy(bias_hbm.at[t],
                                 bias_v.at[gg, pl.ds(r * D, D)], b_sem.at[gg])

        def wait_group(gg):
            for r in range(LANES):
                pltpu.make_async_copy(emb_hbm.at[0],
                                      emb_v.at[gg, pl.ds(r * D, D)],
                                      e_sem.at[gg]).wait()
                pltpu.make_async_copy(bias_hbm.at[0],
                                      bias_v.at[gg, pl.ds(r * D, D)],
                                      b_sem.at[gg]).wait()

        def out_copy(c, par):
            return pltpu.make_async_copy(
                out_v.at[par],
                out_hbm.at[pl.ds(base_b + c * BCHUNK, BCHUNK)],
                o_sem[par])

        lax.fori_loop(0, groups, lambda gg, cr: (fire_group(0, gg), cr)[1], 0)

        def super_body(sg, carry):
            for par in range(OBUF):
                c = sg * OBUF + par

                @pl.when(c >= OBUF)
                def _():
                    out_copy(c - OBUF, par).wait()

                def group_body(gg, carry2):
                    wait_group(gg)
                    sv = inp_v[pl.ds(c * chunk + gg * LANES, LANES)]
                    for r in range(LANES):
                        s = sv[r]
                        tt = gg * LANES + r
                        bb = tt // F
                        f = tt - bb * F
                        for h in range(D // LANES):
                            sl = pl.ds(h * LANES, LANES)
                            x = emb_v[gg, pl.ds(r * D + h * LANES, LANES)] * s \
                                + bias_v[gg, pl.ds(r * D + h * LANES, LANES)]
                            y = jnp.where(x > 0.0, x, jnp.exp(x) - 1.0)
                            out_v[par, bb, f, sl] = y

                    @pl.when(c + 1 < n_chunks)
                    def _():
                        fire_group(c + 1, gg)
                    return carry2

                lax.fori_loop(0, groups, group_body, 0)
                out_copy(c, par).start()
            return carry

        lax.fori_loop(0, n_chunks // OBUF, super_body, 0)
        for par in range(OBUF):
            out_copy(n_chunks - OBUF + par, par).wait()

    return sc_kernel


def kernel(input_tokens, inputs, embeddings, bias):
    B, F = input_tokens.shape
    V, D = embeddings.shape
    BF = B * F
    tok = input_tokens.reshape(BF).astype(jnp.int32)
    inp = inputs.reshape(BF).astype(jnp.float32)
    assert B % (NW * BCHUNK * OBUF) == 0 and D % LANES == 0
    return _build_sc_kernel(B, F, D)(tok, inp, embeddings, bias)


# speedup vs baseline: 1.0545x; 1.0545x over previous
import functools

import jax
import jax.numpy as jnp
from jax import lax
from jax.experimental import pallas as pl
from jax.experimental.pallas import tpu as pltpu
from jax.experimental.pallas import tpu_sc as plsc

LANES = 16
NC = 2
NS = 16
NW = NC * NS
BCHUNK = 8
OBUF = 2


@functools.lru_cache(maxsize=None)
def _build_sc_kernel(B: int, F: int, D: int):
    per_wb = B // NW
    per_w = per_wb * F
    chunk = BCHUNK * F
    groups = chunk // LANES
    assert chunk % LANES == 0
    n_chunks = per_wb // BCHUNK
    assert n_chunks % OBUF == 0
    mesh = plsc.VectorSubcoreMesh(core_axis_name="c", subcore_axis_name="s")

    @functools.partial(
        pl.kernel,
        mesh=mesh,
        out_type=jax.ShapeDtypeStruct((B, F, D), jnp.float32),
        scratch_types=(
            [
                pltpu.VMEM((per_w,), jnp.int32),
                pltpu.VMEM((per_w,), jnp.float32),
                pltpu.VMEM((groups, LANES * D), jnp.float32),
                pltpu.VMEM((groups, LANES * D), jnp.float32),
                pltpu.VMEM((OBUF, BCHUNK, F, D), jnp.float32),
            ]
            + [pltpu.SemaphoreType.DMA((groups,)),
               pltpu.SemaphoreType.DMA((groups,))]
            + [pltpu.SemaphoreType.DMA] * OBUF
        ),
    )
    def sc_kernel(tok_hbm, inp_hbm, emb_hbm, bias_hbm, out_hbm,
                  idx_v, inp_v, emb_v, bias_v, out_v, e_sem, b_sem, *o_sem):
        wid = lax.axis_index("s") * NC + lax.axis_index("c")
        base = wid * per_w
        base_b = wid * per_wb

        pltpu.sync_copy(tok_hbm.at[pl.ds(base, per_w)], idx_v)
        pltpu.sync_copy(inp_hbm.at[pl.ds(base, per_w)], inp_v)

        def fire_group(c, gg):
            tokv = idx_v[pl.ds(c * chunk + gg * LANES, LANES)]
            for r in range(LANES):
                t = tokv[r]
                pltpu.async_copy(emb_hbm.at[t],
                                 emb_v.at[gg, pl.ds(r * D, D)], e_sem.at[gg])
                pltpu.async_copy(bias_hbm.at[t],
                                 bias_v.at[gg, pl.ds(r * D, D)], b_sem.at[gg])

        def wait_group(gg):
            pltpu.make_async_copy(inp_hbm.at[pl.ds(0, LANES * D)],
                                  emb_v.at[gg], e_sem.at[gg]).wait()
            pltpu.make_async_copy(inp_hbm.at[pl.ds(0, LANES * D)],
                                  bias_v.at[gg], b_sem.at[gg]).wait()

        def out_copy(c, par):
            return pltpu.make_async_copy(
                out_v.at[par],
                out_hbm.at[pl.ds(base_b + c * BCHUNK, BCHUNK)],
                o_sem[par])

        lax.fori_loop(0, groups, lambda gg, cr: (fire_group(0, gg), cr)[1], 0)

        def super_body(sg, carry):
            for par in range(OBUF):
                c = sg * OBUF + par

                @pl.when(c >= OBUF)
                def _():
                    out_copy(c - OBUF, par).wait()

                def group_body(gg, carry2):
                    wait_group(gg)
                    sv = inp_v[pl.ds(c * chunk + gg * LANES, LANES)]
                    for r in range(LANES):
                        s = sv[r]
                        tt = gg * LANES + r
                        bb = tt // F
                        f = tt - bb * F
                        for h in range(D // LANES):
                            sl = pl.ds(h * LANES, LANES)
                            x = emb_v[gg, pl.ds(r * D + h * LANES, LANES)] * s \
                                + bias_v[gg, pl.ds(r * D + h * LANES, LANES)]
                            y = jnp.where(x > 0.0, x, jnp.exp(x) - 1.0)
                            out_v[par, bb, f, sl] = y

                    @pl.when(c + 1 < n_chunks)
                    def _():
                        fire_group(c + 1, gg)
                    return carry2

                lax.fori_loop(0, groups, group_body, 0)
                out_copy(c, par).start()
            return carry

        lax.fori_loop(0, n_chunks // OBUF, super_body, 0)
        for par in range(OBUF):
            out_copy(n_chunks - OBUF + par, par).wait()

    return sc_kernel


def kernel(input_tokens, inputs, embeddings, bias):
    B, F = input_tokens.shape
    V, D = embeddings.shape
    BF = B * F
    tok = input_tokens.reshape(BF).astype(jnp.int32)
    inp = inputs.reshape(BF).astype(jnp.float32)
    assert B % (NW * BCHUNK * OBUF) == 0 and D % LANES == 0
    return _build_sc_kernel(B, F, D)(tok, inp, embeddings, bias)


# speedup vs baseline: 1.1234x; 1.0653x over previous
import functools

import jax
import jax.numpy as jnp
from jax import lax
from jax.experimental import pallas as pl
from jax.experimental.pallas import tpu as pltpu
from jax.experimental.pallas import tpu_sc as plsc

LANES = 16
NC = 2
NS = 16
NW = NC * NS
CHUNK = 128
NBUF = 4


@functools.lru_cache(maxsize=None)
def _build_sc_kernel(BF: int, D: int, per_w: int):
    n_chunks = per_w // CHUNK
    assert n_chunks % NBUF == 0
    mesh = plsc.VectorSubcoreMesh(core_axis_name="c", subcore_axis_name="s")

    @functools.partial(
        pl.kernel,
        mesh=mesh,
        out_type=jax.ShapeDtypeStruct((BF, D), jnp.float32),
        compiler_params=pltpu.CompilerParams(use_tc_tiling_on_sc=False),
        scratch_types=(
            [
                pltpu.VMEM((per_w,), jnp.int32),
                pltpu.VMEM((per_w,), jnp.float32),
                pltpu.VMEM((NBUF, CHUNK, D), jnp.float32),
                pltpu.VMEM((NBUF, CHUNK, D), jnp.float32),
                pltpu.VMEM((NBUF, CHUNK, D), jnp.float32),
            ]
            + [pltpu.SemaphoreType.DMA] * (2 * NBUF)
        ),
    )
    def sc_kernel(tok_hbm, inp_hbm, emb_hbm, bias_hbm, out_hbm,
                  idx_v, inp_v, emb_v, bias_v, out_v, *sems):
        g_sem = sems[:NBUF]
        o_sem = sems[NBUF:]
        wid = lax.axis_index("s") * NC + lax.axis_index("c")
        base = wid * per_w

        pltpu.sync_copy(tok_hbm.at[pl.ds(base, per_w)], idx_v)
        pltpu.sync_copy(inp_hbm.at[pl.ds(base, per_w)], inp_v)

        def fire_gathers(c, b):
            idx_slice = idx_v.at[pl.ds(c * CHUNK, CHUNK)]
            pltpu.async_copy(emb_hbm.at[idx_slice], emb_v.at[b], g_sem[b])
            pltpu.async_copy(bias_hbm.at[idx_slice], bias_v.at[b], g_sem[b])

        def wait_gathers(c, b):
            idx_slice = idx_v.at[pl.ds(c * CHUNK, CHUNK)]
            pltpu.make_async_copy(emb_hbm.at[idx_slice], emb_v.at[b],
                                  g_sem[b]).wait()
            pltpu.make_async_copy(bias_hbm.at[idx_slice], bias_v.at[b],
                                  g_sem[b]).wait()

        def out_copy(c, b):
            return pltpu.make_async_copy(
                out_v.at[b], out_hbm.at[pl.ds(base + c * CHUNK, CHUNK)],
                o_sem[b])

        for b in range(NBUF):
            fire_gathers(b, b)

        def ring_body(g, carry):
            for b in range(NBUF):
                c = g * NBUF + b
                wait_gathers(c, b)

                @pl.when(c >= NBUF)
                def _():
                    out_copy(c - NBUF, b).wait()

                def group_body(gr, carry2):
                    row0 = gr * LANES
                    sv = inp_v[pl.ds(c * CHUNK + row0, LANES)]
                    for r in range(LANES):
                        s = sv[r]
                        for h in range(D // LANES):
                            sl = pl.ds(h * LANES, LANES)
                            x = emb_v[b, row0 + r, sl] * s \
                                + bias_v[b, row0 + r, sl]
                            y = jnp.where(x > 0.0, x, jnp.exp(x) - 1.0)
                            out_v[b, row0 + r, sl] = y
                    return carry2

                lax.fori_loop(0, CHUNK // LANES, group_body, 0)
                out_copy(c, b).start()

                @pl.when(c + NBUF < n_chunks)
                def _():
                    fire_gathers(c + NBUF, b)
            return carry

        lax.fori_loop(0, n_chunks // NBUF, ring_body, 0)
        for b in range(NBUF):
            out_copy(n_chunks - NBUF + b, b).wait()

    return sc_kernel


def kernel(input_tokens, inputs, embeddings, bias):
    B, F = input_tokens.shape
    V, D = embeddings.shape
    BF = B * F
    tok = input_tokens.reshape(BF).astype(jnp.int32)
    inp = inputs.reshape(BF).astype(jnp.float32)

    quantum = NW * CHUNK * NBUF
    BFp = ((BF + quantum - 1) // quantum) * quantum
    if BFp != BF:
        tok = jnp.pad(tok, (0, BFp - BF))
        inp = jnp.pad(inp, (0, BFp - BF))

    out = _build_sc_kernel(BFp, D, BFp // NW)(tok, inp, embeddings, bias)
    if BFp != BF:
        out = out[:BF]
    return out.reshape(B, F, D)
